# baseline (speedup 1.0000x reference)
import numpy as np
import jax
import jax.numpy as jnp
from jax.experimental import pallas as pl
from jax.experimental.pallas import tpu as pltpu

T = 8192
HIDDEN = 2048
NHEADS = 16
HEAD_DIM = 128
ROPE_DIM = 64
QLORA = 1536
ROPE_THETA = 10000.0

BM = 512


def _hadamard_permuted():
    h = np.array([[1.0]], dtype=np.float64)
    while h.shape[0] < HEAD_DIM:
        h = np.block([[h, h], [h, -h]])
    inv = np.arange(HEAD_DIM)
    i = np.arange(ROPE_DIM)
    inv[:ROPE_DIM] = np.where(i % 2 == 0, i // 2, ROPE_DIM // 2 + i // 2)
    return h[inv].astype(np.float32)


_H128P = _hadamard_permuted()
_INV_FREQ = (
    1.0 / (ROPE_THETA ** (np.arange(0, ROPE_DIM, 2).astype(np.float32) / ROPE_DIM))
).reshape(1, ROPE_DIM // 2)
_HSCALE = HEAD_DIM ** -0.5


def _indexer_kernel(ql_ref, hid_ref, c_ref, sl_ref, sr_ref, wq_ref, wk_ref,
                    gam_ref, bet_ref, hmat_ref, q_out_ref, k_out_ref,
                    wq_bf_ref, wk_bf_ref):
    @pl.when(pl.program_id(0) == 0)
    def _cache_weights():
        wq_bf_ref[...] = wq_ref[...].astype(jnp.bfloat16)
        wk_bf_ref[...] = wk_ref[...].astype(jnp.bfloat16)

    c1 = c_ref[...]
    sl1 = sl_ref[...]
    sr1 = sr_ref[...]
    hmat = hmat_ref[...]

    def rope_then_h(x):
        xl = jnp.concatenate([x[:, 1:], x[:, :1]], axis=1)
        xr = jnp.concatenate([x[:, -1:], x[:, :-1]], axis=1)
        rot = x * c1 + xl * sl1 + xr * sr1
        return jnp.dot(rot.astype(jnp.bfloat16), hmat,
                       preferred_element_type=jnp.float32)

    k = jnp.dot(hid_ref[...].astype(jnp.bfloat16), wk_bf_ref[...],
                preferred_element_type=jnp.float32)
    mu = jnp.mean(k, axis=1, keepdims=True)
    var = jnp.mean((k - mu) ** 2, axis=1, keepdims=True)
    k = (k - mu) * jax.lax.rsqrt(var + 1e-5) * gam_ref[...] + bet_ref[...]
    k_out_ref[...] = rope_then_h(k)

    q = jnp.dot(ql_ref[...].astype(jnp.bfloat16), wq_bf_ref[...],
                preferred_element_type=jnp.float32)
    for h in range(NHEADS):
        q_out_ref[:, h, :] = rope_then_h(q[:, h * HEAD_DIM:(h + 1) * HEAD_DIM])


@jax.jit
def kernel(q_lora, hidden_states, positions, Wq_b, Wk, k_gamma, k_beta):
    nt = q_lora.shape[0]
    freqs = positions.astype(jnp.float32)[:, None] * jnp.asarray(_INV_FREQ)
    cos = jnp.cos(freqs)
    sin = jnp.sin(freqs)
    z32 = jnp.zeros_like(sin)
    pad = jnp.full((nt, HEAD_DIM - ROPE_DIM), _HSCALE, jnp.float32)
    zpad = jnp.zeros((nt, HEAD_DIM - ROPE_DIM), jnp.float32)
    c_t = jnp.concatenate(
        [jnp.stack([cos, cos], axis=-1).reshape(nt, ROPE_DIM) * _HSCALE, pad],
        axis=1)
    sl_t = jnp.concatenate(
        [jnp.stack([-sin, z32], axis=-1).reshape(nt, ROPE_DIM) * _HSCALE, zpad],
        axis=1)
    sr_t = jnp.concatenate(
        [jnp.stack([z32, sin], axis=-1).reshape(nt, ROPE_DIM) * _HSCALE, zpad],
        axis=1)
    gam = k_gamma.reshape(1, HEAD_DIM)
    bet = k_beta.reshape(1, HEAD_DIM)

    grid = (nt // BM,)
    q2d, key = pl.pallas_call(
        _indexer_kernel,
        grid=grid,
        in_specs=[
            pl.BlockSpec((BM, QLORA), lambda i: (i, 0)),
            pl.BlockSpec((BM, HIDDEN), lambda i: (i, 0)),
            pl.BlockSpec((BM, HEAD_DIM), lambda i: (i, 0)),
            pl.BlockSpec((BM, HEAD_DIM), lambda i: (i, 0)),
            pl.BlockSpec((BM, HEAD_DIM), lambda i: (i, 0)),
            pl.BlockSpec((QLORA, NHEADS * HEAD_DIM), lambda i: (0, 0)),
            pl.BlockSpec((HIDDEN, HEAD_DIM), lambda i: (0, 0)),
            pl.BlockSpec((1, HEAD_DIM), lambda i: (0, 0)),
            pl.BlockSpec((1, HEAD_DIM), lambda i: (0, 0)),
            pl.BlockSpec((HEAD_DIM, HEAD_DIM), lambda i: (0, 0)),
        ],
        out_specs=[
            pl.BlockSpec((BM, NHEADS, HEAD_DIM), lambda i: (i, 0, 0)),
            pl.BlockSpec((BM, HEAD_DIM), lambda i: (i, 0)),
        ],
        out_shape=[
            jax.ShapeDtypeStruct((nt, NHEADS, HEAD_DIM), jnp.float32),
            jax.ShapeDtypeStruct((nt, HEAD_DIM), jnp.float32),
        ],
        scratch_shapes=[
            pltpu.VMEM((QLORA, NHEADS * HEAD_DIM), jnp.bfloat16),
            pltpu.VMEM((HIDDEN, HEAD_DIM), jnp.bfloat16),
        ],
    )(q_lora, hidden_states, c_t, sl_t, sr_t, Wq_b, Wk, gam, bet,
      jnp.asarray(_H128P, dtype=jnp.bfloat16))
    return q2d, key


# speedup vs baseline: 1.1960x; 1.1960x over previous
import numpy as np
import jax
import jax.numpy as jnp
from jax.experimental import pallas as pl
from jax.experimental.pallas import tpu as pltpu

T = 8192
HIDDEN = 2048
NHEADS = 16
HEAD_DIM = 128
ROPE_DIM = 64
QLORA = 1536
ROPE_THETA = 10000.0

BM = 256


def _hadamard_permuted():
    h = np.array([[1.0]], dtype=np.float64)
    while h.shape[0] < HEAD_DIM:
        h = np.block([[h, h], [h, -h]])
    inv = np.arange(HEAD_DIM)
    i = np.arange(ROPE_DIM)
    inv[:ROPE_DIM] = np.where(i % 2 == 0, i // 2, ROPE_DIM // 2 + i // 2)
    return h[inv].astype(np.float32)


_H128P = _hadamard_permuted()
_INV_FREQ = (
    1.0 / (ROPE_THETA ** (np.arange(0, ROPE_DIM, 2).astype(np.float32) / ROPE_DIM))
).reshape(1, ROPE_DIM // 2)
_HSCALE = HEAD_DIM ** -0.5


def _indexer_kernel(ql_ref, hid_ref, c_ref, s_ref, wq_ref, wk_ref,
                    gam_ref, bet_ref, hmat_ref, q_out_ref, k_out_ref,
                    wq_bf_ref, wk_bf_ref):
    @pl.when(pl.program_id(0) == 0)
    def _cache_weights():
        wq_bf_ref[...] = wq_ref[...].astype(jnp.bfloat16)
        wk_bf_ref[...] = wk_ref[...].astype(jnp.bfloat16)

    bm = c_ref.shape[0]
    c64 = c_ref[...]
    s64 = s_ref[...]
    lane = jax.lax.broadcasted_iota(jnp.int32, (bm, ROPE_DIM), 1)
    even = (lane % 2) == 0
    zseg = jnp.zeros((bm, HEAD_DIM - ROPE_DIM), jnp.float32)
    zs = jnp.zeros_like(s64)
    c1 = jnp.concatenate(
        [c64, jnp.full((bm, HEAD_DIM - ROPE_DIM), _HSCALE, jnp.float32)], axis=1)
    sl1 = jnp.concatenate([jnp.where(even, s64, zs), zseg], axis=1)
    sr1 = jnp.concatenate([jnp.where(even, zs, s64), zseg], axis=1)
    hmat = hmat_ref[...]

    def rope_then_h(x):
        xl = jnp.concatenate([x[:, 1:], x[:, :1]], axis=1)
        xr = jnp.concatenate([x[:, -1:], x[:, :-1]], axis=1)
        rot = x * c1 + xl * sl1 + xr * sr1
        return jnp.dot(rot.astype(jnp.bfloat16), hmat,
                       preferred_element_type=jnp.float32)

    k = jnp.dot(hid_ref[...].astype(jnp.bfloat16), wk_bf_ref[...],
                preferred_element_type=jnp.float32)
    mu = jnp.mean(k, axis=1, keepdims=True)
    var = jnp.mean((k - mu) ** 2, axis=1, keepdims=True)
    k = (k - mu) * jax.lax.rsqrt(var + 1e-5) * gam_ref[...] + bet_ref[...]
    k_out_ref[...] = rope_then_h(k)

    q = jnp.dot(ql_ref[...].astype(jnp.bfloat16), wq_bf_ref[...],
                preferred_element_type=jnp.float32)
    for h in range(NHEADS):
        q_out_ref[:, h, :] = rope_then_h(q[:, h * HEAD_DIM:(h + 1) * HEAD_DIM])


@jax.jit
def kernel(q_lora, hidden_states, positions, Wq_b, Wk, k_gamma, k_beta):
    nt = q_lora.shape[0]
    freqs = positions.astype(jnp.float32)[:, None] * jnp.asarray(_INV_FREQ)
    cos = jnp.cos(freqs)
    sin = jnp.sin(freqs)
    c_t = jnp.stack([cos, cos], axis=-1).reshape(nt, ROPE_DIM) * _HSCALE
    s_t = jnp.stack([-sin, sin], axis=-1).reshape(nt, ROPE_DIM) * _HSCALE
    gam = k_gamma.reshape(1, HEAD_DIM)
    bet = k_beta.reshape(1, HEAD_DIM)

    grid = (nt // BM,)
    q2d, key = pl.pallas_call(
        _indexer_kernel,
        grid=grid,
        in_specs=[
            pl.BlockSpec((BM, QLORA), lambda i: (i, 0)),
            pl.BlockSpec((BM, HIDDEN), lambda i: (i, 0)),
            pl.BlockSpec((BM, ROPE_DIM), lambda i: (i, 0)),
            pl.BlockSpec((BM, ROPE_DIM), lambda i: (i, 0)),
            pl.BlockSpec((QLORA, NHEADS * HEAD_DIM), lambda i: (0, 0)),
            pl.BlockSpec((HIDDEN, HEAD_DIM), lambda i: (0, 0)),
            pl.BlockSpec((1, HEAD_DIM), lambda i: (0, 0)),
            pl.BlockSpec((1, HEAD_DIM), lambda i: (0, 0)),
            pl.BlockSpec((HEAD_DIM, HEAD_DIM), lambda i: (0, 0)),
        ],
        out_specs=[
            pl.BlockSpec((BM, NHEADS, HEAD_DIM), lambda i: (i, 0, 0)),
            pl.BlockSpec((BM, HEAD_DIM), lambda i: (i, 0)),
        ],
        out_shape=[
            jax.ShapeDtypeStruct((nt, NHEADS, HEAD_DIM), jnp.float32),
            jax.ShapeDtypeStruct((nt, HEAD_DIM), jnp.float32),
        ],
        scratch_shapes=[
            pltpu.VMEM((QLORA, NHEADS * HEAD_DIM), jnp.bfloat16),
            pltpu.VMEM((HIDDEN, HEAD_DIM), jnp.bfloat16),
        ],
    )(q_lora, hidden_states, c_t, s_t, Wq_b, Wk, gam, bet,
      jnp.asarray(_H128P, dtype=jnp.bfloat16))
    return q2d, key


# speedup vs baseline: 1.2256x; 1.0247x over previous
import numpy as np
import jax
import jax.numpy as jnp
from jax.experimental import pallas as pl
from jax.experimental.pallas import tpu as pltpu

T = 8192
HIDDEN = 2048
NHEADS = 16
HEAD_DIM = 128
ROPE_DIM = 64
QLORA = 1536
ROPE_THETA = 10000.0

BM = 256


def _hadamard_permuted():
    h = np.array([[1.0]], dtype=np.float64)
    while h.shape[0] < HEAD_DIM:
        h = np.block([[h, h], [h, -h]])
    inv = np.arange(HEAD_DIM)
    i = np.arange(ROPE_DIM)
    inv[:ROPE_DIM] = np.where(i % 2 == 0, i // 2, ROPE_DIM // 2 + i // 2)
    return h[inv].astype(np.float32)


_H128P = _hadamard_permuted()
_INV_FREQ = (
    1.0 / (ROPE_THETA ** (np.arange(0, ROPE_DIM, 2).astype(np.float32) / ROPE_DIM))
).reshape(1, ROPE_DIM // 2)
_HSCALE = HEAD_DIM ** -0.5


def _indexer_kernel(ql_ref, hid_ref, pos_ref, ifreq_ref, sgn_ref, wq_ref,
                    wk_ref, gam_ref, bet_ref, hmat_ref, q_out_ref, k_out_ref,
                    wq_bf_ref, wk_bf_ref):
    @pl.when(pl.program_id(0) == 0)
    def _cache_weights():
        wq_bf_ref[...] = wq_ref[...].astype(jnp.bfloat16)
        wk_bf_ref[...] = wk_ref[...].astype(jnp.bfloat16)

    bm = pos_ref.shape[0]
    freqs = pos_ref[...].astype(jnp.float32) * ifreq_ref[...]
    c64 = jnp.cos(freqs) * _HSCALE
    s64 = jnp.sin(freqs) * sgn_ref[...]
    lane = jax.lax.broadcasted_iota(jnp.int32, (bm, ROPE_DIM), 1)
    even = (lane % 2) == 0
    zseg = jnp.zeros((bm, HEAD_DIM - ROPE_DIM), jnp.float32)
    zs = jnp.zeros_like(s64)
    c1 = jnp.concatenate(
        [c64, jnp.full((bm, HEAD_DIM - ROPE_DIM), _HSCALE, jnp.float32)], axis=1)
    sl1 = jnp.concatenate([jnp.where(even, s64, zs), zseg], axis=1)
    sr1 = jnp.concatenate([jnp.where(even, zs, s64), zseg], axis=1)
    hmat = hmat_ref[...]

    def rope_then_h(x):
        xl = jnp.concatenate([x[:, 1:], x[:, :1]], axis=1)
        xr = jnp.concatenate([x[:, -1:], x[:, :-1]], axis=1)
        rot = x * c1 + xl * sl1 + xr * sr1
        return jnp.dot(rot.astype(jnp.bfloat16), hmat,
                       preferred_element_type=jnp.float32)

    k = jnp.dot(hid_ref[...].astype(jnp.bfloat16), wk_bf_ref[...],
                preferred_element_type=jnp.float32)
    mu = jnp.mean(k, axis=1, keepdims=True)
    var = jnp.mean((k - mu) ** 2, axis=1, keepdims=True)
    k = (k - mu) * jax.lax.rsqrt(var + 1e-5) * gam_ref[...] + bet_ref[...]
    k_out_ref[...] = rope_then_h(k)

    q = jnp.dot(ql_ref[...].astype(jnp.bfloat16), wq_bf_ref[...],
                preferred_element_type=jnp.float32)
    for h in range(NHEADS):
        q_out_ref[:, h, :] = rope_then_h(q[:, h * HEAD_DIM:(h + 1) * HEAD_DIM])


@jax.jit
def kernel(q_lora, hidden_states, positions, Wq_b, Wk, k_gamma, k_beta):
    nt = q_lora.shape[0]
    pos2d = positions.reshape(nt, 1)
    ifreq_int = np.repeat(_INV_FREQ[0], 2).reshape(1, ROPE_DIM)
    sgn_int = (np.tile(np.array([-1.0, 1.0], np.float32), ROPE_DIM // 2)
               * _HSCALE).reshape(1, ROPE_DIM)
    gam = k_gamma.reshape(1, HEAD_DIM)
    bet = k_beta.reshape(1, HEAD_DIM)

    grid = (nt // BM,)
    q2d, key = pl.pallas_call(
        _indexer_kernel,
        grid=grid,
        in_specs=[
            pl.BlockSpec((BM, QLORA), lambda i: (i, 0)),
            pl.BlockSpec((BM, HIDDEN), lambda i: (i, 0)),
            pl.BlockSpec((BM, 1), lambda i: (i, 0)),
            pl.BlockSpec((1, ROPE_DIM), lambda i: (0, 0)),
            pl.BlockSpec((1, ROPE_DIM), lambda i: (0, 0)),
            pl.BlockSpec((QLORA, NHEADS * HEAD_DIM), lambda i: (0, 0)),
            pl.BlockSpec((HIDDEN, HEAD_DIM), lambda i: (0, 0)),
            pl.BlockSpec((1, HEAD_DIM), lambda i: (0, 0)),
            pl.BlockSpec((1, HEAD_DIM), lambda i: (0, 0)),
            pl.BlockSpec((HEAD_DIM, HEAD_DIM), lambda i: (0, 0)),
        ],
        out_specs=[
            pl.BlockSpec((BM, NHEADS, HEAD_DIM), lambda i: (i, 0, 0)),
            pl.BlockSpec((BM, HEAD_DIM), lambda i: (i, 0)),
        ],
        out_shape=[
            jax.ShapeDtypeStruct((nt, NHEADS, HEAD_DIM), jnp.float32),
            jax.ShapeDtypeStruct((nt, HEAD_DIM), jnp.float32),
        ],
        scratch_shapes=[
            pltpu.VMEM((QLORA, NHEADS * HEAD_DIM), jnp.bfloat16),
            pltpu.VMEM((HIDDEN, HEAD_DIM), jnp.bfloat16),
        ],
    )(q_lora, hidden_states, pos2d, jnp.asarray(ifreq_int),
      jnp.asarray(sgn_int), Wq_b, Wk, gam, bet,
      jnp.asarray(_H128P, dtype=jnp.bfloat16))
    return q2d, key


# speedup vs baseline: 1.3605x; 1.1101x over previous
import numpy as np
import jax
import jax.numpy as jnp
from jax.experimental import pallas as pl
from jax.experimental.pallas import tpu as pltpu

T = 8192
HIDDEN = 2048
NHEADS = 16
HEAD_DIM = 128
ROPE_DIM = 64
QLORA = 1536
ROPE_THETA = 10000.0

BM = 256


def _hadamard_permuted():
    h = np.array([[1.0]], dtype=np.float64)
    while h.shape[0] < HEAD_DIM:
        h = np.block([[h, h], [h, -h]])
    inv = np.arange(HEAD_DIM)
    i = np.arange(ROPE_DIM)
    inv[:ROPE_DIM] = np.where(i % 2 == 0, i // 2, ROPE_DIM // 2 + i // 2)
    return h[inv].astype(np.float32)


_H128P = _hadamard_permuted()
_INV_FREQ = (
    1.0 / (ROPE_THETA ** (np.arange(0, ROPE_DIM, 2).astype(np.float32) / ROPE_DIM))
).reshape(1, ROPE_DIM // 2)
_HSCALE = HEAD_DIM ** -0.5


def _indexer_kernel(ql_ref, hid_ref, pos_ref, ifreq_ref, sgn_ref, wq_ref,
                    wk_ref, gam_ref, bet_ref, hmat_ref, q_out_ref, k_out_ref,
                    wq_bf_ref, wk_bf_ref):
    @pl.when(pl.program_id(0) == 0)
    def _cache_weights():
        wq_bf_ref[...] = wq_ref[...].astype(jnp.bfloat16)
        wk_bf_ref[...] = wk_ref[...].astype(jnp.bfloat16)

    bm = pos_ref.shape[0]
    freqs = pos_ref[...].astype(jnp.float32) * ifreq_ref[...]
    c64 = jnp.cos(freqs) * _HSCALE
    s64 = jnp.sin(freqs) * sgn_ref[...]
    lane = jax.lax.broadcasted_iota(jnp.int32, (bm, ROPE_DIM), 1)
    even = (lane % 2) == 0
    zseg = jnp.zeros((bm, HEAD_DIM - ROPE_DIM), jnp.float32)
    zs = jnp.zeros_like(s64)
    c1 = jnp.concatenate(
        [c64, jnp.full((bm, HEAD_DIM - ROPE_DIM), _HSCALE, jnp.float32)], axis=1)
    sl1 = jnp.concatenate([jnp.where(even, s64, zs), zseg], axis=1)
    sr1 = jnp.concatenate([jnp.where(even, zs, s64), zseg], axis=1)
    hmat = hmat_ref[...]

    def rope_then_h(x):
        xl = jnp.concatenate([x[:, 1:], x[:, :1]], axis=1)
        xr = jnp.concatenate([x[:, -1:], x[:, :-1]], axis=1)
        rot = x * c1 + xl * sl1 + xr * sr1
        return jnp.dot(rot.astype(jnp.bfloat16), hmat,
                       preferred_element_type=jnp.float32)

    k = jnp.dot(hid_ref[...].astype(jnp.bfloat16), wk_bf_ref[...],
                preferred_element_type=jnp.float32)
    mu = jnp.mean(k, axis=1, keepdims=True)
    var = jnp.mean((k - mu) ** 2, axis=1, keepdims=True)
    k = (k - mu) * jax.lax.rsqrt(var + 1e-5) * gam_ref[...] + bet_ref[...]
    k_out_ref[...] = rope_then_h(k)

    q = jnp.dot(ql_ref[...].astype(jnp.bfloat16), wq_bf_ref[...],
                preferred_element_type=jnp.float32)
    c2 = jnp.concatenate([c1] * NHEADS, axis=1)
    sl2 = jnp.concatenate([sl1] * NHEADS, axis=1)
    sr2 = jnp.concatenate([sr1] * NHEADS, axis=1)
    xl = jnp.concatenate([q[:, 1:], q[:, :1]], axis=1)
    xr = jnp.concatenate([q[:, -1:], q[:, :-1]], axis=1)
    rot = q * c2 + xl * sl2 + xr * sr2
    rot2 = rot.astype(jnp.bfloat16).reshape(bm * NHEADS, HEAD_DIM)
    q_out_ref[...] = jnp.dot(rot2, hmat, preferred_element_type=jnp.float32)


@jax.jit
def kernel(q_lora, hidden_states, positions, Wq_b, Wk, k_gamma, k_beta):
    nt = q_lora.shape[0]
    pos2d = positions.reshape(nt, 1)
    ifreq_int = np.repeat(_INV_FREQ[0], 2).reshape(1, ROPE_DIM)
    sgn_int = (np.tile(np.array([-1.0, 1.0], np.float32), ROPE_DIM // 2)
               * _HSCALE).reshape(1, ROPE_DIM)
    gam = k_gamma.reshape(1, HEAD_DIM)
    bet = k_beta.reshape(1, HEAD_DIM)

    grid = (nt // BM,)
    q2d, key = pl.pallas_call(
        _indexer_kernel,
        grid=grid,
        in_specs=[
            pl.BlockSpec((BM, QLORA), lambda i: (i, 0)),
            pl.BlockSpec((BM, HIDDEN), lambda i: (i, 0)),
            pl.BlockSpec((BM, 1), lambda i: (i, 0)),
            pl.BlockSpec((1, ROPE_DIM), lambda i: (0, 0)),
            pl.BlockSpec((1, ROPE_DIM), lambda i: (0, 0)),
            pl.BlockSpec((QLORA, NHEADS * HEAD_DIM), lambda i: (0, 0)),
            pl.BlockSpec((HIDDEN, HEAD_DIM), lambda i: (0, 0)),
            pl.BlockSpec((1, HEAD_DIM), lambda i: (0, 0)),
            pl.BlockSpec((1, HEAD_DIM), lambda i: (0, 0)),
            pl.BlockSpec((HEAD_DIM, HEAD_DIM), lambda i: (0, 0)),
        ],
        out_specs=[
            pl.BlockSpec((BM * NHEADS, HEAD_DIM), lambda i: (i, 0)),
            pl.BlockSpec((BM, HEAD_DIM), lambda i: (i, 0)),
        ],
        out_shape=[
            jax.ShapeDtypeStruct((nt * NHEADS, HEAD_DIM), jnp.float32),
            jax.ShapeDtypeStruct((nt, HEAD_DIM), jnp.float32),
        ],
        scratch_shapes=[
            pltpu.VMEM((QLORA, NHEADS * HEAD_DIM), jnp.bfloat16),
            pltpu.VMEM((HIDDEN, HEAD_DIM), jnp.bfloat16),
        ],
    )(q_lora, hidden_states, pos2d, jnp.asarray(ifreq_int),
      jnp.asarray(sgn_int), Wq_b, Wk, gam, bet,
      jnp.asarray(_H128P, dtype=jnp.bfloat16))
    return q2d.reshape(nt, NHEADS, HEAD_DIM), key
